# Initial kernel scaffold; baseline (speedup 1.0000x reference)
import jax
import jax.numpy as jnp
from jax.experimental import pallas as pl


def kernel(x, region_mask, kr, ki, fw1, fb1, fw2, fb2, fw3, fb3, pw1, pb1, pw2, pb2, pw3, pb3, aw1, ab1, aw2, ab2, aw3, ab3, bw1, bb1, bw2, bb2, bw3, bb3):
    raise NotImplementedError("write your pallas kernel here")


# speedup vs baseline: 2.7140x; 2.7140x over previous
import functools

import jax
import jax.numpy as jnp
from jax.experimental import pallas as pl
from jax.experimental.pallas import tpu as pltpu

_BLK = 256


def _mlp(xb, w1t, b1, w2t, b2, w3t, b3, act):
    h = act(jnp.dot(xb, w1t, preferred_element_type=jnp.float32) + b1)
    h = act(jnp.dot(h, w2t, preferred_element_type=jnp.float32) + b2)
    return jnp.dot(h, w3t, preferred_element_type=jnp.float32) + b3


def _body(mask_ref, x_ref, mt_ref, *rest):
    (fw1, fb1, fw2, fb2, fw3, fb3,
     pw1, pb1, pw2, pb2, pw3, pb3,
     aw1, ab1, aw2, ab2, aw3, ab3,
     bw1, bb1, bw2, bb2, bw3, bb3, out_ref) = rest
    xb = x_ref[...]
    m = mask_ref[...]
    gelu = lambda v: 0.5 * v * (1.0 + jax.lax.erf(v * 0.7071067811865476))
    relu = lambda v: jnp.maximum(v, 0.0)
    acc = jnp.dot(xb, mt_ref[...], preferred_element_type=jnp.float32)
    o1 = _mlp(xb, fw1[...], fb1[...], fw2[...], fb2[...], fw3[...], fb3[...], gelu)
    acc = jnp.where(m == 1, o1, acc)
    o2 = _mlp(xb, pw1[...], pb1[...], pw2[...], pb2[...], pw3[...], pb3[...], gelu)
    acc = jnp.where(m == 2, o2, acc)
    o3 = _mlp(xb, aw1[...], ab1[...], aw2[...], ab2[...], aw3[...], ab3[...], relu)
    acc = jnp.where(m == 3, o3, acc)
    o4 = _mlp(xb, bw1[...], bb1[...], bw2[...], bb2[...], bw3[...], bb3[...], relu)
    acc = jnp.where(m == 4, o4, acc)
    out_ref[...] = acc


def kernel(x, region_mask, kr, ki, fw1, fb1, fw2, fb2, fw3, fb3, pw1, pb1,
           pw2, pb2, pw3, pb3, aw1, ab1, aw2, ab2, aw3, ab3, bw1, bb1, bw2,
           bb2, bw3, bb3):
    n, d = x.shape
    blk = _BLK
    grid = (n // blk,)

    kmat = kr + 1j * ki
    mt = jnp.real(jnp.fft.ifft(jnp.fft.fft(kmat, axis=1), axis=0)).astype(jnp.float32).T

    mask2 = region_mask.reshape(n, 1)
    weights = (fw1.T, fb1.reshape(1, -1), fw2.T, fb2.reshape(1, -1), fw3.T, fb3.reshape(1, -1),
               pw1.T, pb1.reshape(1, -1), pw2.T, pb2.reshape(1, -1), pw3.T, pb3.reshape(1, -1),
               aw1.T, ab1.reshape(1, -1), aw2.T, ab2.reshape(1, -1), aw3.T, ab3.reshape(1, -1),
               bw1.T, bb1.reshape(1, -1), bw2.T, bb2.reshape(1, -1), bw3.T, bb3.reshape(1, -1))

    full = lambda a: pl.BlockSpec(a.shape, lambda i: (0,) * a.ndim)
    in_specs = [
        pl.BlockSpec((blk, 1), lambda i: (i, 0)),
        pl.BlockSpec((blk, d), lambda i: (i, 0)),
        pl.BlockSpec((d, d), lambda i: (0, 0)),
    ] + [full(w) for w in weights]

    out = pl.pallas_call(
        _body,
        grid=grid,
        in_specs=in_specs,
        out_specs=pl.BlockSpec((blk, d), lambda i: (i, 0)),
        out_shape=jax.ShapeDtypeStruct((n, d), jnp.float32),
    )(mask2, x, mt, *weights)
    return out


# speedup vs baseline: 2.7909x; 1.0283x over previous
import functools

import jax
import jax.numpy as jnp
from jax import lax
from jax.experimental import pallas as pl
from jax.experimental.pallas import tpu as pltpu
from jax.experimental.pallas import tpu_sc as plsc

_BLK = 256
_CHUNK = 32
_CAP = 2048


def _gelu(v):
    return 0.5 * v * (1.0 + jax.lax.erf(v * 0.7071067811865476))


def _relu(v):
    return jnp.maximum(v, 0.0)


def _mlp(xb, w1t, b1, w2t, b2, w3t, b3, act):
    h = act(jnp.dot(xb, w1t, preferred_element_type=jnp.float32) + b1)
    h = act(jnp.dot(h, w2t, preferred_element_type=jnp.float32) + b2)
    return jnp.dot(h, w3t, preferred_element_type=jnp.float32) + b3


def _mlp_select(xb, m, weights):
    (fw1, fb1, fw2, fb2, fw3, fb3,
     pw1, pb1, pw2, pb2, pw3, pb3,
     aw1, ab1, aw2, ab2, aw3, ab3,
     bw1, bb1, bw2, bb2, bw3, bb3) = weights
    acc = _mlp(xb, fw1, fb1, fw2, fb2, fw3, fb3, _gelu)
    o2 = _mlp(xb, pw1, pb1, pw2, pb2, pw3, pb3, _gelu)
    acc = jnp.where(m == 2, o2, acc)
    o3 = _mlp(xb, aw1, ab1, aw2, ab2, aw3, ab3, _relu)
    acc = jnp.where(m == 3, o3, acc)
    o4 = _mlp(xb, bw1, bb1, bw2, bb2, bw3, bb3, _relu)
    return jnp.where(m == 4, o4, acc)


def _tc_mlp_body(mask_ref, x_ref, *rest):
    out_ref = rest[-1]
    weights = tuple(r[...] for r in rest[:-1])
    out_ref[...] = _mlp_select(x_ref[...], mask_ref[...], weights)


def _tc_mono_body(mask_ref, x_ref, mt_ref, *rest):
    out_ref = rest[-1]
    weights = tuple(r[...] for r in rest[:-1])
    acc = _mlp_select(x_ref[...], mask_ref[...], weights)
    o0 = jnp.dot(x_ref[...], mt_ref[...], preferred_element_type=jnp.float32)
    out_ref[...] = jnp.where(mask_ref[...] == 0, o0, acc)


def _tc_matmul_body(xg_ref, mt_ref, og_ref):
    og_ref[...] = jnp.dot(xg_ref[...], mt_ref[...],
                          preferred_element_type=jnp.float32)


def kernel(x, region_mask, kr, ki, fw1, fb1, fw2, fb2, fw3, fb3, pw1, pb1,
           pw2, pb2, pw3, pb3, aw1, ab1, aw2, ab2, aw3, ab3, bw1, bb1, bw2,
           bb2, bw3, bb3):
    n, d = x.shape
    blk = _BLK
    nblocks = n // blk
    cap = _CAP
    capb = cap // blk

    kmat = kr + 1j * ki
    mt = jnp.real(jnp.fft.ifft(jnp.fft.fft(kmat, axis=1), axis=0)).astype(jnp.float32).T

    is0 = region_mask == 0
    n0 = jnp.sum(is0.astype(jnp.int32))
    order = jnp.argsort(jnp.logical_not(is0), stable=True).astype(jnp.int32)
    last0 = order[jnp.maximum(n0 - 1, 0)]
    iota = jnp.arange(cap, dtype=jnp.int32)
    idx = jnp.where(iota < n0, order[:cap], last0)

    mask2 = region_mask.reshape(n, 1)
    weights = (fw1.T, fb1.reshape(1, -1), fw2.T, fb2.reshape(1, -1), fw3.T, fb3.reshape(1, -1),
               pw1.T, pb1.reshape(1, -1), pw2.T, pb2.reshape(1, -1), pw3.T, pb3.reshape(1, -1),
               aw1.T, ab1.reshape(1, -1), aw2.T, ab2.reshape(1, -1), aw3.T, ab3.reshape(1, -1),
               bw1.T, bb1.reshape(1, -1), bw2.T, bb2.reshape(1, -1), bw3.T, bb3.reshape(1, -1))
    full = lambda a: pl.BlockSpec(a.shape, lambda i: (0,) * a.ndim)
    w_specs = [full(w) for w in weights]

    info = plsc.get_sparse_core_info()
    nw = info.num_cores * info.num_subcores
    chunk = _CHUNK
    per_worker = cap // chunk // nw
    assert per_worker * chunk * nw == cap

    mesh = plsc.VectorSubcoreMesh(core_axis_name="c", subcore_axis_name="s")
    sc_scratch = [
        pltpu.VMEM((chunk,), jnp.int32),
        pltpu.VMEM((chunk, d), jnp.float32),
        pltpu.SemaphoreType.DMA,
    ]

    @functools.partial(
        pl.kernel, mesh=mesh,
        out_type=jax.ShapeDtypeStruct((cap, d), jnp.float32),
        scratch_types=sc_scratch,
    )
    def sc_gather(x_hbm, idx_hbm, xg_hbm, idxv, rowsv, sem):
        wid = lax.axis_index("s") * info.num_cores + lax.axis_index("c")

        def body(t, carry):
            base = (t * nw + wid) * chunk
            pltpu.sync_copy(idx_hbm.at[pl.ds(base, chunk)], idxv)
            pltpu.async_copy(x_hbm.at[idxv], rowsv, sem).wait()
            pltpu.sync_copy(rowsv, xg_hbm.at[pl.ds(base, chunk)])
            return carry

        lax.fori_loop(0, per_worker, body, 0)

    @functools.partial(
        pl.kernel, mesh=mesh,
        out_type=(),
        scratch_types=sc_scratch,
    )
    def sc_scatter(og_hbm, idx_hbm, out_hbm, idxv, rowsv, sem):
        wid = lax.axis_index("s") * info.num_cores + lax.axis_index("c")

        def body(t, carry):
            base = (t * nw + wid) * chunk
            pltpu.sync_copy(idx_hbm.at[pl.ds(base, chunk)], idxv)
            pltpu.sync_copy(og_hbm.at[pl.ds(base, chunk)], rowsv)
            pltpu.async_copy(rowsv, out_hbm.at[idxv], sem).wait()
            return carry

        lax.fori_loop(0, per_worker, body, 0)

    def fast_path():
        xg = sc_gather(x, idx)
        og = pl.pallas_call(
            _tc_matmul_body,
            grid=(capb,),
            in_specs=[pl.BlockSpec((blk, d), lambda i: (i, 0)),
                      pl.BlockSpec((d, d), lambda i: (0, 0))],
            out_specs=pl.BlockSpec((blk, d), lambda i: (i, 0)),
            out_shape=jax.ShapeDtypeStruct((cap, d), jnp.float32),
        )(xg, mt)
        mlp_out = pl.pallas_call(
            _tc_mlp_body,
            grid=(nblocks,),
            in_specs=[pl.BlockSpec((blk, 1), lambda i: (i, 0)),
                      pl.BlockSpec((blk, d), lambda i: (i, 0))] + w_specs,
            out_specs=pl.BlockSpec((blk, d), lambda i: (i, 0)),
            out_shape=jax.ShapeDtypeStruct((n, d), jnp.float32),
        )(mask2, x, *weights)
        out_ref = jax.new_ref(mlp_out)
        sc_scatter(og, idx, out_ref)
        return jax.freeze(out_ref)

    def slow_path():
        return pl.pallas_call(
            _tc_mono_body,
            grid=(nblocks,),
            in_specs=[pl.BlockSpec((blk, 1), lambda i: (i, 0)),
                      pl.BlockSpec((blk, d), lambda i: (i, 0)),
                      pl.BlockSpec((d, d), lambda i: (0, 0))] + w_specs,
            out_specs=pl.BlockSpec((blk, d), lambda i: (i, 0)),
            out_shape=jax.ShapeDtypeStruct((n, d), jnp.float32),
        )(mask2, x, mt, *weights)

    return lax.cond((n0 > 0) & (n0 <= cap), fast_path, slow_path)
